# baseline (speedup 1.0000x reference)
import functools

import jax
import jax.numpy as jnp
from jax import lax
from jax.experimental import pallas as pl
from jax.experimental.pallas import tpu as pltpu
from jax.experimental.pallas import tpu_sc as plsc

N = 10000
NPAD = 10240
E = 320000
D = 128

NC = 2
NS = 16
CHUNK = 125
NCHUNK = E // CHUNK
H_CHUNKS = NCHUNK // NS
A_CHUNKS = NCHUNK // (NC * NS)
RPT = NPAD // NS

_sc_mesh = plsc.VectorSubcoreMesh(core_axis_name="c", subcore_axis_name="s")


@functools.partial(
    pl.kernel,
    out_type=(jax.ShapeDtypeStruct((NPAD,), jnp.float32),
              jax.ShapeDtypeStruct((NPAD,), jnp.float32)),
    mesh=_sc_mesh,
    scratch_types=[
        pltpu.VMEM_SHARED((NPAD,), jnp.float32),
        pltpu.VMEM((H_CHUNKS, CHUNK), jnp.int32),
        pltpu.VMEM((CHUNK,), jnp.float32),
        pltpu.VMEM((RPT,), jnp.float32),
        pltpu.SemaphoreType.DMA,
    ],
)
def _degree_kernel(edge_hbm, zeros_hbm, ones_hbm, odeg_hbm, ideg_hbm,
                   deg_sh, ids_v, ones_v, stage_v, sem):
    c = lax.axis_index("c")
    s = lax.axis_index("s")

    pltpu.sync_copy(zeros_hbm, stage_v)
    pltpu.sync_copy(stage_v, deg_sh.at[pl.ds(s * RPT, RPT)])

    pltpu.sync_copy(ones_hbm, ones_v)
    pltpu.sync_copy(edge_hbm.at[c, pl.ds(s * H_CHUNKS, H_CHUNKS), :], ids_v)
    plsc.subcore_barrier()

    def body(j, carry):
        @pl.when(j >= 8)
        def _():
            pltpu.make_async_copy(ones_v, deg_sh.at[ids_v.at[0]], sem).wait()

        pltpu.async_copy(ones_v, deg_sh.at[ids_v.at[j]], sem, add=True)
        return carry

    lax.fori_loop(0, H_CHUNKS, body, 0)

    def drain(j, carry):
        pltpu.make_async_copy(ones_v, deg_sh.at[ids_v.at[0]], sem).wait()
        return carry

    lax.fori_loop(0, 8, drain, 0)
    plsc.subcore_barrier()

    pltpu.sync_copy(deg_sh.at[pl.ds(s * RPT, RPT)], stage_v)

    @pl.when(c == 0)
    def _():
        pltpu.sync_copy(stage_v, odeg_hbm.at[pl.ds(s * RPT, RPT)])

    @pl.when(c == 1)
    def _():
        pltpu.sync_copy(stage_v, ideg_hbm.at[pl.ds(s * RPT, RPT)])


@functools.partial(
    pl.kernel,
    out_type=jax.ShapeDtypeStruct((NC, NPAD, D), jnp.float32),
    mesh=_sc_mesh,
    scratch_types=[
        pltpu.VMEM_SHARED((NPAD, D), jnp.float32),
        pltpu.VMEM((A_CHUNKS // 2, CHUNK), jnp.int32),
        pltpu.VMEM((A_CHUNKS // 2, CHUNK), jnp.int32),
        pltpu.VMEM((CHUNK, D), jnp.float32),
        pltpu.VMEM((CHUNK, D), jnp.float32),
        pltpu.SemaphoreType.DMA,
        pltpu.SemaphoreType.DMA,
        pltpu.SemaphoreType.DMA,
        pltpu.SemaphoreType.DMA,
    ],
)
def _agg_kernel(edge_hbm, h_hbm, zrows_hbm, part_hbm, agg_sh, src_v, dst_v,
                buf0, buf1, sem0, sem1, ssem0, ssem1):
    c = lax.axis_index("c")
    s = lax.axis_index("s")
    w = c * NS + s
    base = w * A_CHUNKS
    half = A_CHUNKS // 2

    pltpu.sync_copy(zrows_hbm, buf0.at[pl.ds(0, 80), :])

    def zbody(k, carry):
        pltpu.async_copy(buf0.at[pl.ds(0, 80), :],
                         agg_sh.at[pl.ds(s * RPT + k * 80, 80), :], sem1)
        return carry

    lax.fori_loop(0, RPT // 80, zbody, 0)

    def zdrain(k, carry):
        pltpu.make_async_copy(buf0.at[pl.ds(0, 80), :],
                              agg_sh.at[pl.ds(s * RPT, 80), :], sem1).wait()
        return carry

    lax.fori_loop(0, RPT // 80, zdrain, 0)
    plsc.subcore_barrier()

    for hlf in range(2):
        pltpu.async_copy(edge_hbm.at[0, pl.ds(base + hlf * half, half), :],
                         src_v, sem0)
        pltpu.async_copy(edge_hbm.at[1, pl.ds(base + hlf * half, half), :],
                         dst_v, sem0)
        pltpu.make_async_copy(edge_hbm.at[0, pl.ds(base, half), :], src_v,
                              sem0).wait()
        pltpu.make_async_copy(edge_hbm.at[1, pl.ds(base, half), :], dst_v,
                              sem0).wait()

        pltpu.async_copy(h_hbm.at[src_v.at[0]], buf0, sem0)

        def body(i, carry):
            j0 = 2 * i
            pltpu.make_async_copy(h_hbm.at[src_v.at[0]], buf0, sem0).wait()
            pltpu.async_copy(buf0, agg_sh.at[dst_v.at[j0]], ssem0, add=True)

            @pl.when(i > 0)
            def _():
                pltpu.make_async_copy(buf1, agg_sh.at[dst_v.at[0]], ssem1).wait()

            pltpu.async_copy(h_hbm.at[src_v.at[j0 + 1]], buf1, sem1)
            pltpu.make_async_copy(h_hbm.at[src_v.at[0]], buf1, sem1).wait()
            pltpu.async_copy(buf1, agg_sh.at[dst_v.at[j0 + 1]], ssem1, add=True)

            pltpu.make_async_copy(buf0, agg_sh.at[dst_v.at[0]], ssem0).wait()

            @pl.when(i < half // 2 - 1)
            def _():
                pltpu.async_copy(h_hbm.at[src_v.at[j0 + 2]], buf0, sem0)

            return carry

        lax.fori_loop(0, half // 2, body, 0)
        pltpu.make_async_copy(buf1, agg_sh.at[dst_v.at[0]], ssem1).wait()

    plsc.subcore_barrier()

    def obody(k, carry):
        pltpu.sync_copy(agg_sh.at[pl.ds(s * RPT + k * 80, 80), :],
                        buf0.at[pl.ds(0, 80), :])
        pltpu.sync_copy(buf0.at[pl.ds(0, 80), :],
                        part_hbm.at[c, pl.ds(s * RPT + k * 80, 80), :])
        return carry

    lax.fori_loop(0, RPT // 80, obody, 0)


def _h_body(x_ref, deg_ref, wr_ref, b_ref, h_ref, res_ref):
    inv = lax.rsqrt(jnp.maximum(deg_ref[...], 1.0))
    h_ref[...] = x_ref[...] * inv
    res_ref[...] = (
        jnp.dot(x_ref[...], wr_ref[...], preferred_element_type=jnp.float32)
        + b_ref[...]
    )


def _out_body(part_ref, indeg_ref, res_ref, wg_ref, out_ref):
    agg = (part_ref[0] + part_ref[1]) * lax.rsqrt(jnp.maximum(indeg_ref[...], 1.0))
    out_ref[...] = (
        jnp.dot(agg, wg_ref[...], preferred_element_type=jnp.float32)
        + res_ref[...]
    )


MBLK = 1000


def kernel(x, edge_index, W_gcn, b_gcn, W_res, b_res):
    edge_r = edge_index.astype(jnp.int32).reshape(2, NCHUNK, CHUNK)
    zeros640 = jnp.zeros((RPT,), jnp.float32)
    ones125 = jnp.ones((CHUNK,), jnp.float32)
    zrows = jnp.zeros((80, D), jnp.float32)

    out_deg, in_deg = _degree_kernel(edge_r, zeros640, ones125)
    out_deg2d = out_deg.reshape(NPAD, 1)
    in_deg2d = in_deg.reshape(NPAD, 1)

    bias = (b_gcn + b_res).reshape(1, D)
    h, res = pl.pallas_call(
        _h_body,
        grid=(N // MBLK,),
        in_specs=[
            pl.BlockSpec((MBLK, D), lambda i: (i, 0)),
            pl.BlockSpec((MBLK, 1), lambda i: (i, 0)),
            pl.BlockSpec((D, D), lambda i: (0, 0)),
            pl.BlockSpec((1, D), lambda i: (0, 0)),
        ],
        out_specs=[
            pl.BlockSpec((MBLK, D), lambda i: (i, 0)),
            pl.BlockSpec((MBLK, D), lambda i: (i, 0)),
        ],
        out_shape=[
            jax.ShapeDtypeStruct((N, D), jnp.float32),
            jax.ShapeDtypeStruct((N, D), jnp.float32),
        ],
    )(x, out_deg2d, W_res, bias)

    part = _agg_kernel(edge_r, h, zrows)

    out = pl.pallas_call(
        _out_body,
        grid=(N // MBLK,),
        in_specs=[
            pl.BlockSpec((NC, MBLK, D), lambda i: (0, i, 0)),
            pl.BlockSpec((MBLK, 1), lambda i: (i, 0)),
            pl.BlockSpec((MBLK, D), lambda i: (i, 0)),
            pl.BlockSpec((D, D), lambda i: (0, 0)),
        ],
        out_specs=pl.BlockSpec((MBLK, D), lambda i: (i, 0)),
        out_shape=jax.ShapeDtypeStruct((N, D), jnp.float32),
    )(part, in_deg2d, res, W_gcn)
    return out


# speedup vs baseline: 1.0285x; 1.0285x over previous
import functools

import jax
import jax.numpy as jnp
from jax import lax
from jax.experimental import pallas as pl
from jax.experimental.pallas import tpu as pltpu
from jax.experimental.pallas import tpu_sc as plsc

N = 10000
NPAD = 10240
E = 320000
D = 128

NC = 2
NS = 16
CHUNK = 125
NCHUNK = E // CHUNK
H_CHUNKS = NCHUNK // NS
A_CHUNKS = NCHUNK // (NC * NS)
RPT = NPAD // NS

_sc_mesh = plsc.VectorSubcoreMesh(core_axis_name="c", subcore_axis_name="s")


@functools.partial(
    pl.kernel,
    out_type=(jax.ShapeDtypeStruct((NPAD,), jnp.float32),
              jax.ShapeDtypeStruct((NPAD,), jnp.float32)),
    mesh=_sc_mesh,
    scratch_types=[
        pltpu.VMEM_SHARED((NPAD,), jnp.float32),
        pltpu.VMEM((H_CHUNKS, CHUNK), jnp.int32),
        pltpu.VMEM((CHUNK,), jnp.float32),
        pltpu.VMEM((RPT,), jnp.float32),
        pltpu.SemaphoreType.DMA,
    ],
)
def _degree_kernel(edge_hbm, zeros_hbm, ones_hbm, odeg_hbm, ideg_hbm,
                   deg_sh, ids_v, ones_v, stage_v, sem):
    c = lax.axis_index("c")
    s = lax.axis_index("s")

    pltpu.sync_copy(zeros_hbm, stage_v)
    pltpu.sync_copy(stage_v, deg_sh.at[pl.ds(s * RPT, RPT)])

    pltpu.sync_copy(ones_hbm, ones_v)
    pltpu.sync_copy(edge_hbm.at[c, pl.ds(s * H_CHUNKS, H_CHUNKS), :], ids_v)
    plsc.subcore_barrier()

    def body(j, carry):
        @pl.when(j >= 8)
        def _():
            pltpu.make_async_copy(ones_v, deg_sh.at[ids_v.at[0]], sem).wait()

        pltpu.async_copy(ones_v, deg_sh.at[ids_v.at[j]], sem, add=True)
        return carry

    lax.fori_loop(0, H_CHUNKS, body, 0)

    def drain(j, carry):
        pltpu.make_async_copy(ones_v, deg_sh.at[ids_v.at[0]], sem).wait()
        return carry

    lax.fori_loop(0, 8, drain, 0)
    plsc.subcore_barrier()

    pltpu.sync_copy(deg_sh.at[pl.ds(s * RPT, RPT)], stage_v)

    @pl.when(c == 0)
    def _():
        pltpu.sync_copy(stage_v, odeg_hbm.at[pl.ds(s * RPT, RPT)])

    @pl.when(c == 1)
    def _():
        pltpu.sync_copy(stage_v, ideg_hbm.at[pl.ds(s * RPT, RPT)])


@functools.partial(
    pl.kernel,
    out_type=jax.ShapeDtypeStruct((NC, NPAD, D), jnp.float32),
    mesh=_sc_mesh,
    scratch_types=[
        pltpu.VMEM_SHARED((NPAD, D), jnp.float32),
        pltpu.VMEM((A_CHUNKS // 2, CHUNK), jnp.int32),
        pltpu.VMEM((A_CHUNKS // 2, CHUNK), jnp.int32),
        pltpu.VMEM((CHUNK, D), jnp.float32),
        pltpu.VMEM((CHUNK, D), jnp.float32),
        pltpu.SemaphoreType.DMA,
        pltpu.SemaphoreType.DMA,
        pltpu.SemaphoreType.DMA,
        pltpu.SemaphoreType.DMA,
    ],
)
def _agg_kernel(edge_hbm, h_hbm, zrows_hbm, part_hbm, agg_sh, src_v, dst_v,
                buf0, buf1, sem0, sem1, ssem0, ssem1):
    c = lax.axis_index("c")
    s = lax.axis_index("s")
    w = c * NS + s
    base = w * A_CHUNKS
    half = A_CHUNKS // 2

    pltpu.sync_copy(zrows_hbm, buf0.at[pl.ds(0, 80), :])

    def zbody(k, carry):
        pltpu.async_copy(buf0.at[pl.ds(0, 80), :],
                         agg_sh.at[pl.ds(s * RPT + k * 80, 80), :], sem1)
        return carry

    lax.fori_loop(0, RPT // 80, zbody, 0)

    def zdrain(k, carry):
        pltpu.make_async_copy(buf0.at[pl.ds(0, 80), :],
                              agg_sh.at[pl.ds(s * RPT, 80), :], sem1).wait()
        return carry

    lax.fori_loop(0, RPT // 80, zdrain, 0)
    plsc.subcore_barrier()

    for hlf in range(2):
        pltpu.async_copy(edge_hbm.at[0, pl.ds(base + hlf * half, half), :],
                         src_v, sem0)
        pltpu.async_copy(edge_hbm.at[1, pl.ds(base + hlf * half, half), :],
                         dst_v, sem0)
        pltpu.make_async_copy(edge_hbm.at[0, pl.ds(base, half), :], src_v,
                              sem0).wait()
        pltpu.make_async_copy(edge_hbm.at[1, pl.ds(base, half), :], dst_v,
                              sem0).wait()

        pltpu.async_copy(h_hbm.at[src_v.at[0]], buf0, sem0)

        def body(i, carry):
            j0 = 2 * i
            pltpu.make_async_copy(h_hbm.at[src_v.at[0]], buf0, sem0).wait()
            pltpu.async_copy(buf0, agg_sh.at[dst_v.at[j0]], ssem0, add=True)

            @pl.when(i > 0)
            def _():
                pltpu.make_async_copy(buf1, agg_sh.at[dst_v.at[0]], ssem1).wait()

            pltpu.async_copy(h_hbm.at[src_v.at[j0 + 1]], buf1, sem1)
            pltpu.make_async_copy(h_hbm.at[src_v.at[0]], buf1, sem1).wait()
            pltpu.async_copy(buf1, agg_sh.at[dst_v.at[j0 + 1]], ssem1, add=True)

            pltpu.make_async_copy(buf0, agg_sh.at[dst_v.at[0]], ssem0).wait()

            @pl.when(i < half // 2 - 1)
            def _():
                pltpu.async_copy(h_hbm.at[src_v.at[j0 + 2]], buf0, sem0)

            return carry

        lax.fori_loop(0, half // 2, body, 0)
        pltpu.make_async_copy(buf1, agg_sh.at[dst_v.at[0]], ssem1).wait()

    plsc.subcore_barrier()

    def obody(k, carry):
        pltpu.sync_copy(agg_sh.at[pl.ds(s * RPT + k * 80, 80), :],
                        buf0.at[pl.ds(0, 80), :])
        pltpu.sync_copy(buf0.at[pl.ds(0, 80), :],
                        part_hbm.at[c, pl.ds(s * RPT + k * 80, 80), :])
        return carry

    lax.fori_loop(0, RPT // 80, obody, 0)


def _h_body(x_ref, deg_ref, wr_ref, b_ref, h_ref, res_ref):
    inv = lax.rsqrt(jnp.maximum(deg_ref[...], 1.0))
    h_ref[...] = x_ref[...] * inv
    res_ref[...] = (
        jnp.dot(x_ref[...], wr_ref[...], preferred_element_type=jnp.float32)
        + b_ref[...]
    )


def _out_body(part_ref, indeg_ref, res_ref, wg_ref, out_ref):
    agg = (part_ref[0] + part_ref[1]) * lax.rsqrt(jnp.maximum(indeg_ref[...], 1.0))
    out_ref[...] = (
        jnp.dot(agg, wg_ref[...], preferred_element_type=jnp.float32)
        + res_ref[...]
    )


MBLK = 2000


def kernel(x, edge_index, W_gcn, b_gcn, W_res, b_res):
    edge_r = edge_index.astype(jnp.int32).reshape(2, NCHUNK, CHUNK)
    zeros640 = jnp.zeros((RPT,), jnp.float32)
    ones125 = jnp.ones((CHUNK,), jnp.float32)
    zrows = jnp.zeros((80, D), jnp.float32)

    out_deg, in_deg = _degree_kernel(edge_r, zeros640, ones125)
    out_deg2d = out_deg.reshape(NPAD, 1)
    in_deg2d = in_deg.reshape(NPAD, 1)

    bias = (b_gcn + b_res).reshape(1, D)
    h, res = pl.pallas_call(
        _h_body,
        grid=(N // MBLK,),
        in_specs=[
            pl.BlockSpec((MBLK, D), lambda i: (i, 0)),
            pl.BlockSpec((MBLK, 1), lambda i: (i, 0)),
            pl.BlockSpec((D, D), lambda i: (0, 0)),
            pl.BlockSpec((1, D), lambda i: (0, 0)),
        ],
        out_specs=[
            pl.BlockSpec((MBLK, D), lambda i: (i, 0)),
            pl.BlockSpec((MBLK, D), lambda i: (i, 0)),
        ],
        out_shape=[
            jax.ShapeDtypeStruct((N, D), jnp.float32),
            jax.ShapeDtypeStruct((N, D), jnp.float32),
        ],
    )(x, out_deg2d, W_res, bias)

    part = _agg_kernel(edge_r, h, zrows)

    out = pl.pallas_call(
        _out_body,
        grid=(N // MBLK,),
        in_specs=[
            pl.BlockSpec((NC, MBLK, D), lambda i: (0, i, 0)),
            pl.BlockSpec((MBLK, 1), lambda i: (i, 0)),
            pl.BlockSpec((MBLK, D), lambda i: (i, 0)),
            pl.BlockSpec((D, D), lambda i: (0, 0)),
        ],
        out_specs=pl.BlockSpec((MBLK, D), lambda i: (i, 0)),
        out_shape=jax.ShapeDtypeStruct((N, D), jnp.float32),
    )(part, in_deg2d, res, W_gcn)
    return out


# speedup vs baseline: 1.0487x; 1.0196x over previous
import functools

import jax
import jax.numpy as jnp
from jax import lax
from jax.experimental import pallas as pl
from jax.experimental.pallas import tpu as pltpu
from jax.experimental.pallas import tpu_sc as plsc

N = 10000
NPAD = 10240
E = 320000
D = 128

NC = 2
NS = 16
CHUNK = 125
NCHUNK = E // CHUNK
H_CHUNKS = NCHUNK // NS
A_CHUNKS = NCHUNK // (NC * NS)
RPT = NPAD // NS

_sc_mesh = plsc.VectorSubcoreMesh(core_axis_name="c", subcore_axis_name="s")


@functools.partial(
    pl.kernel,
    out_type=(jax.ShapeDtypeStruct((NPAD,), jnp.float32),
              jax.ShapeDtypeStruct((NPAD,), jnp.float32)),
    mesh=_sc_mesh,
    scratch_types=[
        pltpu.VMEM_SHARED((NPAD,), jnp.float32),
        pltpu.VMEM((H_CHUNKS, CHUNK), jnp.int32),
        pltpu.VMEM((CHUNK,), jnp.float32),
        pltpu.VMEM((RPT,), jnp.float32),
        pltpu.SemaphoreType.DMA,
    ],
)
def _degree_kernel(edge_hbm, zeros_hbm, ones_hbm, odeg_hbm, ideg_hbm,
                   deg_sh, ids_v, ones_v, stage_v, sem):
    c = lax.axis_index("c")
    s = lax.axis_index("s")

    pltpu.sync_copy(zeros_hbm, stage_v)
    pltpu.sync_copy(stage_v, deg_sh.at[pl.ds(s * RPT, RPT)])

    pltpu.sync_copy(ones_hbm, ones_v)
    pltpu.sync_copy(edge_hbm.at[c, pl.ds(s * H_CHUNKS, H_CHUNKS), :], ids_v)
    plsc.subcore_barrier()

    def body(j, carry):
        @pl.when(j >= 8)
        def _():
            pltpu.make_async_copy(ones_v, deg_sh.at[ids_v.at[0]], sem).wait()

        pltpu.async_copy(ones_v, deg_sh.at[ids_v.at[j]], sem, add=True)
        return carry

    lax.fori_loop(0, H_CHUNKS, body, 0)

    def drain(j, carry):
        pltpu.make_async_copy(ones_v, deg_sh.at[ids_v.at[0]], sem).wait()
        return carry

    lax.fori_loop(0, 8, drain, 0)
    plsc.subcore_barrier()

    pltpu.sync_copy(deg_sh.at[pl.ds(s * RPT, RPT)], stage_v)

    @pl.when(c == 0)
    def _():
        pltpu.sync_copy(stage_v, odeg_hbm.at[pl.ds(s * RPT, RPT)])

    @pl.when(c == 1)
    def _():
        pltpu.sync_copy(stage_v, ideg_hbm.at[pl.ds(s * RPT, RPT)])


@functools.partial(
    pl.kernel,
    out_type=jax.ShapeDtypeStruct((NC, NPAD, D), jnp.float32),
    mesh=_sc_mesh,
    scratch_types=[
        pltpu.VMEM_SHARED((NPAD, D), jnp.float32),
        pltpu.VMEM((A_CHUNKS // 2, CHUNK), jnp.int32),
        pltpu.VMEM((A_CHUNKS // 2, CHUNK), jnp.int32),
        pltpu.VMEM((CHUNK, D), jnp.float32),
        pltpu.VMEM((CHUNK, D), jnp.float32),
        pltpu.SemaphoreType.DMA,
        pltpu.SemaphoreType.DMA,
        pltpu.SemaphoreType.DMA,
        pltpu.SemaphoreType.DMA,
    ],
)
def _agg_kernel(edge_hbm, h_hbm, zrows_hbm, part_hbm, agg_sh, src_v, dst_v,
                buf0, buf1, sem0, sem1, ssem0, ssem1):
    c = lax.axis_index("c")
    s = lax.axis_index("s")
    w = c * NS + s
    base = w * A_CHUNKS
    half = A_CHUNKS // 2

    pltpu.sync_copy(zrows_hbm, buf1.at[pl.ds(0, 80), :])

    def zbody(k, carry):
        pltpu.async_copy(buf1.at[pl.ds(0, 80), :],
                         agg_sh.at[pl.ds(s * RPT + k * 80, 80), :], sem1)
        return carry

    lax.fori_loop(0, RPT // 80, zbody, 0)

    pltpu.async_copy(edge_hbm.at[0, pl.ds(base, half), :], src_v, sem0)
    pltpu.async_copy(edge_hbm.at[1, pl.ds(base, half), :], dst_v, sem0)
    pltpu.make_async_copy(edge_hbm.at[0, pl.ds(base, half), :], src_v,
                          sem0).wait()
    pltpu.make_async_copy(edge_hbm.at[1, pl.ds(base, half), :], dst_v,
                          sem0).wait()
    pltpu.async_copy(h_hbm.at[src_v.at[0]], buf0, sem0)

    def zdrain(k, carry):
        pltpu.make_async_copy(buf1.at[pl.ds(0, 80), :],
                              agg_sh.at[pl.ds(s * RPT, 80), :], sem1).wait()
        return carry

    lax.fori_loop(0, RPT // 80, zdrain, 0)
    plsc.subcore_barrier()

    for hlf in range(2):
        if hlf > 0:
            pltpu.async_copy(edge_hbm.at[0, pl.ds(base + hlf * half, half), :],
                             src_v, sem0)
            pltpu.async_copy(edge_hbm.at[1, pl.ds(base + hlf * half, half), :],
                             dst_v, sem0)
            pltpu.make_async_copy(edge_hbm.at[0, pl.ds(base, half), :], src_v,
                                  sem0).wait()
            pltpu.make_async_copy(edge_hbm.at[1, pl.ds(base, half), :], dst_v,
                                  sem0).wait()
            pltpu.async_copy(h_hbm.at[src_v.at[0]], buf0, sem0)

        def body(i, carry):
            j0 = 2 * i
            pltpu.make_async_copy(h_hbm.at[src_v.at[0]], buf0, sem0).wait()
            pltpu.async_copy(buf0, agg_sh.at[dst_v.at[j0]], ssem0, add=True)

            @pl.when(i > 0)
            def _():
                pltpu.make_async_copy(buf1, agg_sh.at[dst_v.at[0]], ssem1).wait()

            pltpu.async_copy(h_hbm.at[src_v.at[j0 + 1]], buf1, sem1)
            pltpu.make_async_copy(h_hbm.at[src_v.at[0]], buf1, sem1).wait()
            pltpu.async_copy(buf1, agg_sh.at[dst_v.at[j0 + 1]], ssem1, add=True)

            pltpu.make_async_copy(buf0, agg_sh.at[dst_v.at[0]], ssem0).wait()

            @pl.when(i < half // 2 - 1)
            def _():
                pltpu.async_copy(h_hbm.at[src_v.at[j0 + 2]], buf0, sem0)

            return carry

        lax.fori_loop(0, half // 2, body, 0)
        pltpu.make_async_copy(buf1, agg_sh.at[dst_v.at[0]], ssem1).wait()

    plsc.subcore_barrier()

    pltpu.async_copy(agg_sh.at[pl.ds(s * RPT, 80), :],
                     buf0.at[pl.ds(0, 80), :], sem0)

    def obody(k2, carry):
        k = 2 * k2
        pltpu.make_async_copy(agg_sh.at[pl.ds(s * RPT, 80), :],
                              buf0.at[pl.ds(0, 80), :], sem0).wait()
        pltpu.async_copy(agg_sh.at[pl.ds(s * RPT + (k + 1) * 80, 80), :],
                         buf1.at[pl.ds(0, 80), :], sem1)
        pltpu.sync_copy(buf0.at[pl.ds(0, 80), :],
                        part_hbm.at[c, pl.ds(s * RPT + k * 80, 80), :])
        pltpu.make_async_copy(agg_sh.at[pl.ds(s * RPT, 80), :],
                              buf1.at[pl.ds(0, 80), :], sem1).wait()

        @pl.when(k2 < RPT // 160 - 1)
        def _():
            pltpu.async_copy(agg_sh.at[pl.ds(s * RPT + (k + 2) * 80, 80), :],
                             buf0.at[pl.ds(0, 80), :], sem0)

        pltpu.sync_copy(buf1.at[pl.ds(0, 80), :],
                        part_hbm.at[c, pl.ds(s * RPT + (k + 1) * 80, 80), :])
        return carry

    lax.fori_loop(0, RPT // 160, obody, 0)


def _h_body(x_ref, deg_ref, wr_ref, b_ref, h_ref, res_ref):
    inv = lax.rsqrt(jnp.maximum(deg_ref[...], 1.0))
    h_ref[...] = x_ref[...] * inv
    res_ref[...] = (
        jnp.dot(x_ref[...], wr_ref[...], preferred_element_type=jnp.float32)
        + b_ref[...]
    )


def _out_body(part_ref, indeg_ref, res_ref, wg_ref, out_ref):
    agg = (part_ref[0] + part_ref[1]) * lax.rsqrt(jnp.maximum(indeg_ref[...], 1.0))
    out_ref[...] = (
        jnp.dot(agg, wg_ref[...], preferred_element_type=jnp.float32)
        + res_ref[...]
    )


MBLK = 2000


def kernel(x, edge_index, W_gcn, b_gcn, W_res, b_res):
    edge_r = edge_index.astype(jnp.int32).reshape(2, NCHUNK, CHUNK)
    zeros640 = jnp.zeros((RPT,), jnp.float32)
    ones125 = jnp.ones((CHUNK,), jnp.float32)
    zrows = jnp.zeros((80, D), jnp.float32)

    out_deg, in_deg = _degree_kernel(edge_r, zeros640, ones125)
    out_deg2d = out_deg.reshape(NPAD, 1)
    in_deg2d = in_deg.reshape(NPAD, 1)

    bias = (b_gcn + b_res).reshape(1, D)
    h, res = pl.pallas_call(
        _h_body,
        grid=(N // MBLK,),
        in_specs=[
            pl.BlockSpec((MBLK, D), lambda i: (i, 0)),
            pl.BlockSpec((MBLK, 1), lambda i: (i, 0)),
            pl.BlockSpec((D, D), lambda i: (0, 0)),
            pl.BlockSpec((1, D), lambda i: (0, 0)),
        ],
        out_specs=[
            pl.BlockSpec((MBLK, D), lambda i: (i, 0)),
            pl.BlockSpec((MBLK, D), lambda i: (i, 0)),
        ],
        out_shape=[
            jax.ShapeDtypeStruct((N, D), jnp.float32),
            jax.ShapeDtypeStruct((N, D), jnp.float32),
        ],
    )(x, out_deg2d, W_res, bias)

    part = _agg_kernel(edge_r, h, zrows)

    out = pl.pallas_call(
        _out_body,
        grid=(N // MBLK,),
        in_specs=[
            pl.BlockSpec((NC, MBLK, D), lambda i: (0, i, 0)),
            pl.BlockSpec((MBLK, 1), lambda i: (i, 0)),
            pl.BlockSpec((MBLK, D), lambda i: (i, 0)),
            pl.BlockSpec((D, D), lambda i: (0, 0)),
        ],
        out_specs=pl.BlockSpec((MBLK, D), lambda i: (i, 0)),
        out_shape=jax.ShapeDtypeStruct((N, D), jnp.float32),
    )(part, in_deg2d, res, W_gcn)
    return out


# speedup vs baseline: 1.0581x; 1.0090x over previous
import functools

import jax
import jax.numpy as jnp
from jax import lax
from jax.experimental import pallas as pl
from jax.experimental.pallas import tpu as pltpu
from jax.experimental.pallas import tpu_sc as plsc

N = 10000
NPAD = 10240
E = 320000
D = 128

NC = 2
NS = 16
CHUNK = 125
NCHUNK = E // CHUNK
H_CHUNKS = NCHUNK // NS
A_CHUNKS = NCHUNK // (NC * NS)
RPT = NPAD // NS

_sc_mesh = plsc.VectorSubcoreMesh(core_axis_name="c", subcore_axis_name="s")


@functools.partial(
    pl.kernel,
    out_type=(jax.ShapeDtypeStruct((NPAD,), jnp.float32),
              jax.ShapeDtypeStruct((NPAD,), jnp.float32)),
    mesh=_sc_mesh,
    scratch_types=[
        pltpu.VMEM_SHARED((NPAD,), jnp.float32),
        pltpu.VMEM((H_CHUNKS, CHUNK), jnp.int32),
        pltpu.VMEM((CHUNK,), jnp.float32),
        pltpu.VMEM((RPT,), jnp.float32),
        pltpu.SemaphoreType.DMA,
    ],
)
def _degree_kernel(edge_hbm, zeros_hbm, ones_hbm, odeg_hbm, ideg_hbm,
                   deg_sh, ids_v, ones_v, stage_v, sem):
    c = lax.axis_index("c")
    s = lax.axis_index("s")

    pltpu.async_copy(edge_hbm.at[c, pl.ds(s * H_CHUNKS, H_CHUNKS), :], ids_v, sem)
    pltpu.sync_copy(zeros_hbm, stage_v)
    pltpu.sync_copy(stage_v, deg_sh.at[pl.ds(s * RPT, RPT)])
    pltpu.sync_copy(ones_hbm, ones_v)
    pltpu.make_async_copy(edge_hbm.at[c, pl.ds(s * H_CHUNKS, H_CHUNKS), :],
                          ids_v, sem).wait()
    plsc.subcore_barrier()

    def body(j, carry):
        @pl.when(j >= 8)
        def _():
            pltpu.make_async_copy(ones_v, deg_sh.at[ids_v.at[0]], sem).wait()

        pltpu.async_copy(ones_v, deg_sh.at[ids_v.at[j]], sem, add=True)
        return carry

    lax.fori_loop(0, H_CHUNKS, body, 0)

    def drain(j, carry):
        pltpu.make_async_copy(ones_v, deg_sh.at[ids_v.at[0]], sem).wait()
        return carry

    lax.fori_loop(0, 8, drain, 0)
    plsc.subcore_barrier()

    pltpu.sync_copy(deg_sh.at[pl.ds(s * RPT, RPT)], stage_v)

    @pl.when(c == 0)
    def _():
        pltpu.sync_copy(stage_v, odeg_hbm.at[pl.ds(s * RPT, RPT)])

    @pl.when(c == 1)
    def _():
        pltpu.sync_copy(stage_v, ideg_hbm.at[pl.ds(s * RPT, RPT)])


@functools.partial(
    pl.kernel,
    out_type=jax.ShapeDtypeStruct((NC, NPAD, D), jnp.float32),
    mesh=_sc_mesh,
    scratch_types=[
        pltpu.VMEM_SHARED((NPAD, D), jnp.float32),
        pltpu.VMEM((A_CHUNKS // 2, CHUNK), jnp.int32),
        pltpu.VMEM((A_CHUNKS // 2, CHUNK), jnp.int32),
        pltpu.VMEM((CHUNK, D), jnp.float32),
        pltpu.VMEM((CHUNK, D), jnp.float32),
        pltpu.SemaphoreType.DMA,
        pltpu.SemaphoreType.DMA,
        pltpu.SemaphoreType.DMA,
        pltpu.SemaphoreType.DMA,
    ],
)
def _agg_kernel(edge_hbm, h_hbm, zrows_hbm, part_hbm, agg_sh, src_v, dst_v,
                buf0, buf1, sem0, sem1, ssem0, ssem1):
    c = lax.axis_index("c")
    s = lax.axis_index("s")
    w = c * NS + s
    base = w * A_CHUNKS
    half = A_CHUNKS // 2

    pltpu.sync_copy(zrows_hbm, buf1.at[pl.ds(0, 80), :])

    def zbody(k, carry):
        pltpu.async_copy(buf1.at[pl.ds(0, 80), :],
                         agg_sh.at[pl.ds(s * RPT + k * 80, 80), :], sem1)
        return carry

    lax.fori_loop(0, RPT // 80, zbody, 0)

    pltpu.async_copy(edge_hbm.at[0, pl.ds(base, half), :], src_v, sem0)
    pltpu.async_copy(edge_hbm.at[1, pl.ds(base, half), :], dst_v, sem0)
    pltpu.make_async_copy(edge_hbm.at[0, pl.ds(base, half), :], src_v,
                          sem0).wait()
    pltpu.make_async_copy(edge_hbm.at[1, pl.ds(base, half), :], dst_v,
                          sem0).wait()
    pltpu.async_copy(h_hbm.at[src_v.at[0]], buf0, sem0)

    def zdrain(k, carry):
        pltpu.make_async_copy(buf1.at[pl.ds(0, 80), :],
                              agg_sh.at[pl.ds(s * RPT, 80), :], sem1).wait()
        return carry

    lax.fori_loop(0, RPT // 80, zdrain, 0)
    plsc.subcore_barrier()

    for hlf in range(2):
        if hlf > 0:
            pltpu.async_copy(edge_hbm.at[0, pl.ds(base + hlf * half, half), :],
                             src_v, sem0)
            pltpu.async_copy(edge_hbm.at[1, pl.ds(base + hlf * half, half), :],
                             dst_v, sem0)
            pltpu.make_async_copy(edge_hbm.at[0, pl.ds(base, half), :], src_v,
                                  sem0).wait()
            pltpu.make_async_copy(edge_hbm.at[1, pl.ds(base, half), :], dst_v,
                                  sem0).wait()
            pltpu.async_copy(h_hbm.at[src_v.at[0]], buf0, sem0)

        def body(i, carry):
            j0 = 2 * i
            pltpu.make_async_copy(h_hbm.at[src_v.at[0]], buf0, sem0).wait()
            pltpu.async_copy(buf0, agg_sh.at[dst_v.at[j0]], ssem0, add=True)

            @pl.when(i > 0)
            def _():
                pltpu.make_async_copy(buf1, agg_sh.at[dst_v.at[0]], ssem1).wait()

            pltpu.async_copy(h_hbm.at[src_v.at[j0 + 1]], buf1, sem1)
            pltpu.make_async_copy(h_hbm.at[src_v.at[0]], buf1, sem1).wait()
            pltpu.async_copy(buf1, agg_sh.at[dst_v.at[j0 + 1]], ssem1, add=True)

            pltpu.make_async_copy(buf0, agg_sh.at[dst_v.at[0]], ssem0).wait()

            @pl.when(i < half // 2 - 1)
            def _():
                pltpu.async_copy(h_hbm.at[src_v.at[j0 + 2]], buf0, sem0)

            return carry

        lax.fori_loop(0, half // 2, body, 0)
        pltpu.make_async_copy(buf1, agg_sh.at[dst_v.at[0]], ssem1).wait()

    plsc.subcore_barrier()

    pltpu.async_copy(agg_sh.at[pl.ds(s * RPT, 80), :],
                     buf0.at[pl.ds(0, 80), :], sem0)

    def obody(k2, carry):
        k = 2 * k2
        pltpu.make_async_copy(agg_sh.at[pl.ds(s * RPT, 80), :],
                              buf0.at[pl.ds(0, 80), :], sem0).wait()
        pltpu.async_copy(agg_sh.at[pl.ds(s * RPT + (k + 1) * 80, 80), :],
                         buf1.at[pl.ds(0, 80), :], sem1)
        pltpu.sync_copy(buf0.at[pl.ds(0, 80), :],
                        part_hbm.at[c, pl.ds(s * RPT + k * 80, 80), :])
        pltpu.make_async_copy(agg_sh.at[pl.ds(s * RPT, 80), :],
                              buf1.at[pl.ds(0, 80), :], sem1).wait()

        @pl.when(k2 < RPT // 160 - 1)
        def _():
            pltpu.async_copy(agg_sh.at[pl.ds(s * RPT + (k + 2) * 80, 80), :],
                             buf0.at[pl.ds(0, 80), :], sem0)

        pltpu.sync_copy(buf1.at[pl.ds(0, 80), :],
                        part_hbm.at[c, pl.ds(s * RPT + (k + 1) * 80, 80), :])
        return carry

    lax.fori_loop(0, RPT // 160, obody, 0)


def _h_body(x_ref, deg_ref, wr_ref, b_ref, h_ref, res_ref):
    inv = lax.rsqrt(jnp.maximum(deg_ref[...], 1.0))
    h_ref[...] = x_ref[...] * inv
    res_ref[...] = (
        jnp.dot(x_ref[...], wr_ref[...], preferred_element_type=jnp.float32)
        + b_ref[...]
    )


def _out_body(part_ref, indeg_ref, res_ref, wg_ref, out_ref):
    agg = (part_ref[0] + part_ref[1]) * lax.rsqrt(jnp.maximum(indeg_ref[...], 1.0))
    out_ref[...] = (
        jnp.dot(agg, wg_ref[...], preferred_element_type=jnp.float32)
        + res_ref[...]
    )


MBLK = 2000


def kernel(x, edge_index, W_gcn, b_gcn, W_res, b_res):
    edge_r = edge_index.astype(jnp.int32).reshape(2, NCHUNK, CHUNK)
    zeros640 = jnp.zeros((RPT,), jnp.float32)
    ones125 = jnp.ones((CHUNK,), jnp.float32)
    zrows = jnp.zeros((80, D), jnp.float32)

    out_deg, in_deg = _degree_kernel(edge_r, zeros640, ones125)
    out_deg2d = out_deg.reshape(NPAD, 1)
    in_deg2d = in_deg.reshape(NPAD, 1)

    bias = (b_gcn + b_res).reshape(1, D)
    h, res = pl.pallas_call(
        _h_body,
        grid=(N // MBLK,),
        in_specs=[
            pl.BlockSpec((MBLK, D), lambda i: (i, 0)),
            pl.BlockSpec((MBLK, 1), lambda i: (i, 0)),
            pl.BlockSpec((D, D), lambda i: (0, 0)),
            pl.BlockSpec((1, D), lambda i: (0, 0)),
        ],
        out_specs=[
            pl.BlockSpec((MBLK, D), lambda i: (i, 0)),
            pl.BlockSpec((MBLK, D), lambda i: (i, 0)),
        ],
        out_shape=[
            jax.ShapeDtypeStruct((N, D), jnp.float32),
            jax.ShapeDtypeStruct((N, D), jnp.float32),
        ],
    )(x, out_deg2d, W_res, bias)

    part = _agg_kernel(edge_r, h, zrows)

    out = pl.pallas_call(
        _out_body,
        grid=(N // MBLK,),
        in_specs=[
            pl.BlockSpec((NC, MBLK, D), lambda i: (0, i, 0)),
            pl.BlockSpec((MBLK, 1), lambda i: (i, 0)),
            pl.BlockSpec((MBLK, D), lambda i: (i, 0)),
            pl.BlockSpec((D, D), lambda i: (0, 0)),
        ],
        out_specs=pl.BlockSpec((MBLK, D), lambda i: (i, 0)),
        out_shape=jax.ShapeDtypeStruct((N, D), jnp.float32),
    )(part, in_deg2d, res, W_gcn)
    return out


# speedup vs baseline: 1.0602x; 1.0019x over previous
import functools

import jax
import jax.numpy as jnp
from jax import lax
from jax.experimental import pallas as pl
from jax.experimental.pallas import tpu as pltpu
from jax.experimental.pallas import tpu_sc as plsc

N = 10000
NPAD = 10240
E = 320000
D = 128

NC = 2
NS = 16
CHUNK = 125
NCHUNK = E // CHUNK
H_CHUNKS = NCHUNK // NS
A_CHUNKS = NCHUNK // (NC * NS)
RPT = NPAD // NS

_sc_mesh = plsc.VectorSubcoreMesh(core_axis_name="c", subcore_axis_name="s")


@functools.partial(
    pl.kernel,
    out_type=(jax.ShapeDtypeStruct((NPAD,), jnp.float32),
              jax.ShapeDtypeStruct((NPAD,), jnp.float32)),
    mesh=_sc_mesh,
    scratch_types=[
        pltpu.VMEM_SHARED((NPAD,), jnp.float32),
        pltpu.VMEM((H_CHUNKS, CHUNK), jnp.int32),
        pltpu.VMEM((CHUNK,), jnp.float32),
        pltpu.VMEM((RPT,), jnp.float32),
        pltpu.SemaphoreType.DMA,
    ],
)
def _degree_kernel(edge_hbm, zeros_hbm, ones_hbm, odeg_hbm, ideg_hbm,
                   deg_sh, ids_v, ones_v, stage_v, sem):
    c = lax.axis_index("c")
    s = lax.axis_index("s")

    pltpu.async_copy(edge_hbm.at[c, pl.ds(s * H_CHUNKS, H_CHUNKS), :], ids_v, sem)
    pltpu.sync_copy(zeros_hbm, stage_v)
    pltpu.sync_copy(stage_v, deg_sh.at[pl.ds(s * RPT, RPT)])
    pltpu.sync_copy(ones_hbm, ones_v)
    pltpu.make_async_copy(edge_hbm.at[c, pl.ds(s * H_CHUNKS, H_CHUNKS), :],
                          ids_v, sem).wait()
    plsc.subcore_barrier()

    def body(j, carry):
        @pl.when(j >= 16)
        def _():
            pltpu.make_async_copy(ones_v, deg_sh.at[ids_v.at[0]], sem).wait()

        pltpu.async_copy(ones_v, deg_sh.at[ids_v.at[j]], sem, add=True)
        return carry

    lax.fori_loop(0, H_CHUNKS, body, 0)

    def drain(j, carry):
        pltpu.make_async_copy(ones_v, deg_sh.at[ids_v.at[0]], sem).wait()
        return carry

    lax.fori_loop(0, 16, drain, 0)
    plsc.subcore_barrier()

    pltpu.sync_copy(deg_sh.at[pl.ds(s * RPT, RPT)], stage_v)

    @pl.when(c == 0)
    def _():
        pltpu.sync_copy(stage_v, odeg_hbm.at[pl.ds(s * RPT, RPT)])

    @pl.when(c == 1)
    def _():
        pltpu.sync_copy(stage_v, ideg_hbm.at[pl.ds(s * RPT, RPT)])


@functools.partial(
    pl.kernel,
    out_type=jax.ShapeDtypeStruct((NC, NPAD, D), jnp.float32),
    mesh=_sc_mesh,
    scratch_types=[
        pltpu.VMEM_SHARED((NPAD, D), jnp.float32),
        pltpu.VMEM((A_CHUNKS // 2, CHUNK), jnp.int32),
        pltpu.VMEM((A_CHUNKS // 2, CHUNK), jnp.int32),
        pltpu.VMEM((CHUNK, D), jnp.float32),
        pltpu.VMEM((CHUNK, D), jnp.float32),
        pltpu.SemaphoreType.DMA,
        pltpu.SemaphoreType.DMA,
        pltpu.SemaphoreType.DMA,
        pltpu.SemaphoreType.DMA,
    ],
)
def _agg_kernel(edge_hbm, h_hbm, zrows_hbm, part_hbm, agg_sh, src_v, dst_v,
                buf0, buf1, sem0, sem1, ssem0, ssem1):
    c = lax.axis_index("c")
    s = lax.axis_index("s")
    w = c * NS + s
    base = w * A_CHUNKS
    half = A_CHUNKS // 2

    pltpu.sync_copy(zrows_hbm, buf1.at[pl.ds(0, 80), :])

    def zbody(k, carry):
        pltpu.async_copy(buf1.at[pl.ds(0, 80), :],
                         agg_sh.at[pl.ds(s * RPT + k * 80, 80), :], sem1)
        return carry

    lax.fori_loop(0, RPT // 80, zbody, 0)

    pltpu.async_copy(edge_hbm.at[0, pl.ds(base, half), :], src_v, sem0)
    pltpu.async_copy(edge_hbm.at[1, pl.ds(base, half), :], dst_v, sem0)
    pltpu.make_async_copy(edge_hbm.at[0, pl.ds(base, half), :], src_v,
                          sem0).wait()
    pltpu.make_async_copy(edge_hbm.at[1, pl.ds(base, half), :], dst_v,
                          sem0).wait()
    pltpu.async_copy(h_hbm.at[src_v.at[0]], buf0, sem0)

    def zdrain(k, carry):
        pltpu.make_async_copy(buf1.at[pl.ds(0, 80), :],
                              agg_sh.at[pl.ds(s * RPT, 80), :], sem1).wait()
        return carry

    lax.fori_loop(0, RPT // 80, zdrain, 0)
    plsc.subcore_barrier()

    for hlf in range(2):
        if hlf > 0:
            pltpu.async_copy(edge_hbm.at[0, pl.ds(base + hlf * half, half), :],
                             src_v, sem0)
            pltpu.async_copy(edge_hbm.at[1, pl.ds(base + hlf * half, half), :],
                             dst_v, sem0)
            pltpu.make_async_copy(edge_hbm.at[0, pl.ds(base, half), :], src_v,
                                  sem0).wait()
            pltpu.make_async_copy(edge_hbm.at[1, pl.ds(base, half), :], dst_v,
                                  sem0).wait()
            pltpu.async_copy(h_hbm.at[src_v.at[0]], buf0, sem0)

        def body(i, carry):
            j0 = 2 * i
            pltpu.make_async_copy(h_hbm.at[src_v.at[0]], buf0, sem0).wait()
            pltpu.async_copy(buf0, agg_sh.at[dst_v.at[j0]], ssem0, add=True)

            @pl.when(i > 0)
            def _():
                pltpu.make_async_copy(buf1, agg_sh.at[dst_v.at[0]], ssem1).wait()

            pltpu.async_copy(h_hbm.at[src_v.at[j0 + 1]], buf1, sem1)
            pltpu.make_async_copy(h_hbm.at[src_v.at[0]], buf1, sem1).wait()
            pltpu.async_copy(buf1, agg_sh.at[dst_v.at[j0 + 1]], ssem1, add=True)

            pltpu.make_async_copy(buf0, agg_sh.at[dst_v.at[0]], ssem0).wait()

            @pl.when(i < half // 2 - 1)
            def _():
                pltpu.async_copy(h_hbm.at[src_v.at[j0 + 2]], buf0, sem0)

            return carry

        lax.fori_loop(0, half // 2, body, 0)
        pltpu.make_async_copy(buf1, agg_sh.at[dst_v.at[0]], ssem1).wait()

    plsc.subcore_barrier()

    pltpu.async_copy(agg_sh.at[pl.ds(s * RPT, 80), :],
                     buf0.at[pl.ds(0, 80), :], sem0)

    def obody(k2, carry):
        k = 2 * k2
        pltpu.make_async_copy(agg_sh.at[pl.ds(s * RPT, 80), :],
                              buf0.at[pl.ds(0, 80), :], sem0).wait()
        pltpu.async_copy(agg_sh.at[pl.ds(s * RPT + (k + 1) * 80, 80), :],
                         buf1.at[pl.ds(0, 80), :], sem1)
        pltpu.sync_copy(buf0.at[pl.ds(0, 80), :],
                        part_hbm.at[c, pl.ds(s * RPT + k * 80, 80), :])
        pltpu.make_async_copy(agg_sh.at[pl.ds(s * RPT, 80), :],
                              buf1.at[pl.ds(0, 80), :], sem1).wait()

        @pl.when(k2 < RPT // 160 - 1)
        def _():
            pltpu.async_copy(agg_sh.at[pl.ds(s * RPT + (k + 2) * 80, 80), :],
                             buf0.at[pl.ds(0, 80), :], sem0)

        pltpu.sync_copy(buf1.at[pl.ds(0, 80), :],
                        part_hbm.at[c, pl.ds(s * RPT + (k + 1) * 80, 80), :])
        return carry

    lax.fori_loop(0, RPT // 160, obody, 0)


def _h_body(x_ref, deg_ref, wr_ref, b_ref, h_ref, res_ref):
    inv = lax.rsqrt(jnp.maximum(deg_ref[...], 1.0))
    h_ref[...] = x_ref[...] * inv
    res_ref[...] = (
        jnp.dot(x_ref[...], wr_ref[...], preferred_element_type=jnp.float32)
        + b_ref[...]
    )


def _out_body(part_ref, indeg_ref, res_ref, wg_ref, out_ref):
    agg = (part_ref[0] + part_ref[1]) * lax.rsqrt(jnp.maximum(indeg_ref[...], 1.0))
    out_ref[...] = (
        jnp.dot(agg, wg_ref[...], preferred_element_type=jnp.float32)
        + res_ref[...]
    )


MBLK = 2000


def kernel(x, edge_index, W_gcn, b_gcn, W_res, b_res):
    edge_r = edge_index.astype(jnp.int32).reshape(2, NCHUNK, CHUNK)
    zeros640 = jnp.zeros((RPT,), jnp.float32)
    ones125 = jnp.ones((CHUNK,), jnp.float32)
    zrows = jnp.zeros((80, D), jnp.float32)

    out_deg, in_deg = _degree_kernel(edge_r, zeros640, ones125)
    out_deg2d = out_deg.reshape(NPAD, 1)
    in_deg2d = in_deg.reshape(NPAD, 1)

    bias = (b_gcn + b_res).reshape(1, D)
    h, res = pl.pallas_call(
        _h_body,
        grid=(N // MBLK,),
        in_specs=[
            pl.BlockSpec((MBLK, D), lambda i: (i, 0)),
            pl.BlockSpec((MBLK, 1), lambda i: (i, 0)),
            pl.BlockSpec((D, D), lambda i: (0, 0)),
            pl.BlockSpec((1, D), lambda i: (0, 0)),
        ],
        out_specs=[
            pl.BlockSpec((MBLK, D), lambda i: (i, 0)),
            pl.BlockSpec((MBLK, D), lambda i: (i, 0)),
        ],
        out_shape=[
            jax.ShapeDtypeStruct((N, D), jnp.float32),
            jax.ShapeDtypeStruct((N, D), jnp.float32),
        ],
    )(x, out_deg2d, W_res, bias)

    part = _agg_kernel(edge_r, h, zrows)

    out = pl.pallas_call(
        _out_body,
        grid=(N // MBLK,),
        in_specs=[
            pl.BlockSpec((NC, MBLK, D), lambda i: (0, i, 0)),
            pl.BlockSpec((MBLK, 1), lambda i: (i, 0)),
            pl.BlockSpec((MBLK, D), lambda i: (i, 0)),
            pl.BlockSpec((D, D), lambda i: (0, 0)),
        ],
        out_specs=pl.BlockSpec((MBLK, D), lambda i: (i, 0)),
        out_shape=jax.ShapeDtypeStruct((N, D), jnp.float32),
    )(part, in_deg2d, res, W_gcn)
    return out
